# baseline (speedup 1.0000x reference)
import jax
import jax.numpy as jnp
from jax import lax
from jax.experimental import pallas as pl
from jax.experimental.pallas import tpu as pltpu
from jax.experimental.pallas import tpu_sc as plsc

N_NODES = 100000
MEM_DIM = 64
BATCH = 16384
HALF_LIFE = 40.0
LN2 = 0.69314718

NC, NS, L = 2, 16, 16
NW = NC * NS
ROWS_W = 3128
CP_A = 3032
CP_B = 96
CHUNK = 256
N_GROUPS = BATCH // L


def _sc_body(ids_hbm, ns_hbm, ts_hbm, mem_hbm, lut_hbm,
             memout_hbm, tsout_hbm,
             ids_v, ts_v, lut_v, claim_v, list_v,
             bidx_v, nidx_v, alpha_v, tsn_v,
             old_v, nsr_v, new_v, s_in, s_cp, sem0, sem1):
    wid = lax.axis_index("c") * NS + lax.axis_index("s")
    lo = wid * ROWS_W
    nrows = jnp.minimum(ROWS_W, N_NODES - lo)
    lane = lax.iota(jnp.int32, L)
    ob = jnp.minimum(lo + CP_A, N_NODES - CP_B)
    ob8 = pl.multiple_of(ob, 8)
    lo8 = pl.multiple_of(lo, 8)

    c_ids = pltpu.async_copy(ids_hbm, ids_v, s_in)
    c_ts = pltpu.async_copy(ts_hbm, ts_v, s_in)
    c_lv_a = pltpu.async_copy(lut_hbm.at[pl.ds(lo8, CP_A)],
                              lut_v.at[pl.ds(0, CP_A)], s_in)
    c_lv_b = pltpu.async_copy(lut_hbm.at[pl.ds(ob8, CP_B)],
                              lut_v.at[pl.ds(CP_A, CP_B)], s_in)
    c_m_a = pltpu.async_copy(mem_hbm.at[pl.ds(lo, CP_A), :],
                             memout_hbm.at[pl.ds(lo, CP_A), :], s_cp)
    c_m_b = pltpu.async_copy(mem_hbm.at[pl.ds(ob, CP_B), :],
                             memout_hbm.at[pl.ds(ob, CP_B), :], s_cp)
    c_l_a = pltpu.async_copy(lut_hbm.at[pl.ds(lo8, CP_A)],
                             tsout_hbm.at[pl.ds(lo8, CP_A)], s_cp)
    c_l_b = pltpu.async_copy(lut_hbm.at[pl.ds(ob8, CP_B)],
                             tsout_hbm.at[pl.ds(ob8, CP_B)], s_cp)

    c_ids.wait()
    c_ts.wait()

    hi = lo + nrows

    def claim_grp(g, _):
        ids16 = ids_v[pl.ds(g * L, L)]
        own = (ids16 >= lo) & (ids16 < hi)
        bvec = g * L + lane
        idsl = jnp.where(own, ids16 - lo, 0)
        _, lastm = plsc.scan_count(ids16, mask=own)
        plsc.store_scatter(claim_v, [idsl], bvec, mask=own & lastm)
        return 0
    lax.fori_loop(0, N_GROUPS, claim_grp, 0, unroll=4)

    def compact_grp(g, ptr):
        ids16 = ids_v[pl.ds(g * L, L)]
        own = (ids16 >= lo) & (ids16 < hi)
        bvec = g * L + lane
        idsl = jnp.where(own, ids16 - lo, 0)
        w = plsc.load_gather(claim_v, [idsl])
        win = own & (w == bvec)
        c = plsc.cumsum(win.astype(jnp.int32))
        plsc.store_scatter(list_v, [ptr + c - 1], bvec, mask=win)
        return ptr + c[L - 1]
    nwin = lax.fori_loop(0, N_GROUPS, compact_grp, jnp.int32(0), unroll=4)

    nchunks = (nwin + CHUNK - 1) // CHUNK

    def pad_grp(g, _):
        pos16 = g * L + lane
        cur = plsc.load_gather(list_v, [jnp.minimum(pos16, BATCH - 1)])
        first = plsc.load_gather(list_v, [jnp.zeros((L,), jnp.int32)])
        plsc.store_scatter(list_v, [pos16], jnp.where(pos16 < nwin, cur, first))
        return 0
    lax.fori_loop(nwin // L, (nchunks * CHUNK) // L, pad_grp, 0)

    c_lv_a.wait()
    c_lv_b.wait()
    c_m_a.wait()
    c_m_b.wait()
    c_l_a.wait()
    c_l_b.wait()

    def chunk_body(ck, _):
        base = ck * CHUNK
        for g in range(CHUNK // L):
            pos = base + g * L + lane
            bv = plsc.load_gather(list_v, [pos])
            nv = plsc.load_gather(ids_v, [bv])
            tsv = plsc.load_gather(ts_v, [bv])
            prev = plsc.load_gather(lut_v, [nv - lo])
            dt = jnp.maximum(tsv - prev, 0.0)
            av = jnp.exp(-LN2 * dt / HALF_LIFE)
            sl = pl.ds(g * L, L)
            bidx_v[sl] = bv
            nidx_v[sl] = nv
            alpha_v[sl] = av
            tsn_v[sl] = tsv
        cold = pltpu.async_copy(mem_hbm.at[nidx_v], old_v, sem0)
        cns = pltpu.async_copy(ns_hbm.at[bidx_v], nsr_v, sem1)
        cold.wait()
        cns.wait()

        def row_body(r, _):
            a = plsc.load_gather(alpha_v, [jnp.full((L,), r, jnp.int32)])
            for c in range(MEM_DIM // L):
                sl = pl.ds(c * L, L)
                new_v[r, sl] = a * old_v[r, sl] + (1.0 - a) * nsr_v[r, sl]
            return 0
        lax.fori_loop(0, CHUNK, row_body, 0, unroll=2)

        s1 = pltpu.async_copy(new_v, memout_hbm.at[nidx_v], sem0)
        s2 = pltpu.async_copy(tsn_v, tsout_hbm.at[nidx_v], sem1)
        s1.wait()
        s2.wait()
        return 0
    lax.fori_loop(0, nchunks, chunk_body, 0)


@jax.jit
def kernel(node_ids, new_states, ts, memory, last_update_ts):
    mesh = plsc.VectorSubcoreMesh(core_axis_name="c", subcore_axis_name="s",
                                  num_cores=NC, num_subcores=NS)
    f = pl.kernel(
        _sc_body,
        out_type=(jax.ShapeDtypeStruct((N_NODES, MEM_DIM), jnp.float32),
                  jax.ShapeDtypeStruct((N_NODES,), jnp.float32)),
        mesh=mesh,
        scratch_types=(
            pltpu.VMEM((BATCH,), jnp.int32),
            pltpu.VMEM((BATCH,), jnp.float32),
            pltpu.VMEM((ROWS_W,), jnp.float32),
            pltpu.VMEM((ROWS_W,), jnp.int32),
            pltpu.VMEM((BATCH + CHUNK,), jnp.int32),
            pltpu.VMEM((CHUNK,), jnp.int32),
            pltpu.VMEM((CHUNK,), jnp.int32),
            pltpu.VMEM((CHUNK,), jnp.float32),
            pltpu.VMEM((CHUNK,), jnp.float32),
            pltpu.VMEM((CHUNK, MEM_DIM), jnp.float32),
            pltpu.VMEM((CHUNK, MEM_DIM), jnp.float32),
            pltpu.VMEM((CHUNK, MEM_DIM), jnp.float32),
            pltpu.SemaphoreType.DMA,
            pltpu.SemaphoreType.DMA,
            pltpu.SemaphoreType.DMA,
            pltpu.SemaphoreType.DMA,
        ),
        compiler_params=pltpu.CompilerParams(needs_layout_passes=False,
                                             use_tc_tiling_on_sc=False),
    )
    return f(node_ids, new_states, ts, memory, last_update_ts)


# speedup vs baseline: 4.0737x; 4.0737x over previous
import jax
import jax.numpy as jnp
from jax import lax
from jax.experimental import pallas as pl
from jax.experimental.pallas import tpu as pltpu
from jax.experimental.pallas import tpu_sc as plsc

N_NODES = 100000
MEM_DIM = 64
BATCH = 16384
HALF_LIFE = 40.0
LN2 = 0.69314718

NC, NS, L = 2, 16, 16
BLK = 1024
TAIL0 = 99968
CLAIM = 4 * BLK
N_GROUPS = BATCH // L
SB = 64


def _sc_body(ids_hbm, ns_hbm, ts_hbm, memt_hbm, lut_hbm, tail_hbm,
             memout_hbm, tsout_hbm, tailout_hbm,
             ids_v, lut_v, claim_v, listb_v, listc_v, slab_v,
             ns_v, nst_v, colb_v, avb_v, mb_v, nsidx_v, nb_v, tsb_v, tail_v,
             s_in, s_ts, s_sl, s_ns, s_sc):
    wid = lax.axis_index("c") * NS + lax.axis_index("s")
    lane = lax.iota(jnp.int32, L)

    c_ids = pltpu.async_copy(ids_hbm, ids_v, s_in)
    c_lut = []
    for k in range(3):
        off = pl.multiple_of((wid + 32 * k) * BLK, BLK)
        c_lut.append(pltpu.async_copy(lut_hbm.at[pl.ds(off, BLK)],
                                      lut_v.at[pl.ds(k * BLK, BLK)], s_in))

    @pl.when(wid == 0)
    def _():
        pltpu.sync_copy(lut_hbm.at[pl.ds(96 * BLK, BLK)],
                        lut_v.at[pl.ds(3 * BLK, BLK)])

    @pl.when(wid == 1)
    def _():
        pltpu.sync_copy(lut_hbm.at[pl.ds(97 * BLK, 672)],
                        lut_v.at[pl.ds(3 * BLK, 672)])


    def memset_grp(g, _):
        claim_v[pl.ds(g * L, L)] = jnp.full((L,), -1, jnp.int32)
        return 0
    lax.fori_loop(0, CLAIM // L, memset_grp, 0, unroll=4)

    c_ids.wait()

    def claim_grp(g, _):
        ids16 = ids_v[pl.ds(g * L, L)]
        own = ((ids16 >> 10) & 31) == wid
        li = ((ids16 >> 15) << 10) | (ids16 & 1023)
        _, lastm = plsc.scan_count(ids16, mask=own)
        plsc.store_scatter(claim_v, [li], g * L + lane, mask=own & lastm)
        return 0
    lax.fori_loop(0, N_GROUPS, claim_grp, 0, unroll=4)

    def compact_grp(g, ptr):
        cl16 = claim_v[pl.ds(g * L, L)]
        valid = cl16 != -1
        c = plsc.cumsum(valid.astype(jnp.int32))
        pos = ptr + c - 1
        plsc.store_scatter(listb_v, [pos], cl16, mask=valid)
        plsc.store_scatter(listc_v, [pos], g * L + lane, mask=valid)
        return ptr + c[L - 1]
    p = [jnp.int32(0)]
    for k in range(4):
        p.append(lax.fori_loop(64 * k, 64 * (k + 1), compact_grp, p[k],
                               unroll=4))

    for c in c_lut:
        c.wait()
    t_cp = []
    for k in range(3):
        off = pl.multiple_of((wid + 32 * k) * BLK, BLK)
        t_cp.append(pltpu.async_copy(lut_v.at[pl.ds(k * BLK, BLK)],
                                     tsout_hbm.at[pl.ds(off, BLK)], s_ts))

    @pl.when(wid == 0)
    def _():
        pltpu.sync_copy(lut_v.at[pl.ds(3 * BLK, BLK)],
                        tsout_hbm.at[pl.ds(96 * BLK, BLK)])

    @pl.when(wid == 1)
    def _():
        pltpu.sync_copy(lut_v.at[pl.ds(3 * BLK, 672)],
                        tsout_hbm.at[pl.ds(97 * BLK, 672)])
    for c in t_cp:
        c.wait()

    is0 = wid == 0
    is1 = wid == 1

    def make_block_pass(sz):
        def block_pass(off, plo, phi):
            rds = [pltpu.async_copy(memt_hbm.at[j, pl.ds(off, sz)],
                                    slab_v.at[pl.ds(j * sz, sz)], s_sl)
                   for j in range(MEM_DIM)]
            for d in rds:
                d.wait()

            def sub_batch(sb, _):
                base = sb * SB
                for g in range(SB // L):
                    pos = base + g * L + lane
                    posc = jnp.clip(pos, plo, phi - 1)
                    bp = plsc.load_gather(listb_v, [posc])
                    li = plsc.load_gather(listc_v, [posc])
                    col = li & 1023
                    m = (pos >= plo) & (pos < phi) & (col < sz)
                    nglob = ((wid + ((li >> 10) << 5)) << 10) | col
                    sl = pl.ds(g * L, L)
                    colb_v[sl] = col
                    mb_v[sl] = m.astype(jnp.int32)
                    nsidx_v[sl] = bp
                    nb_v[sl] = nglob
                c_ns = pltpu.async_copy(ns_hbm.at[nsidx_v], ns_v, s_ns)
                c_tb = pltpu.async_copy(ts_hbm.at[nsidx_v], tsb_v, s_ns)
                c_ns.wait()
                c_tb.wait()
                for g in range(SB // L):
                    sl = pl.ds(g * L, L)
                    pos = base + g * L + lane
                    posc = jnp.clip(pos, plo, phi - 1)
                    li = plsc.load_gather(listc_v, [posc])
                    prev = plsc.load_gather(lut_v, [li])
                    dt = jnp.maximum(tsb_v[sl] - prev, 0.0)
                    avb_v[sl] = jnp.exp(-LN2 * dt / HALF_LIFE)
                c_sc = pltpu.async_copy(tsb_v, tsout_hbm.at[nb_v], s_sc)

                def tr(w, _):
                    for jc in range(MEM_DIM // L):
                        v16 = ns_v[w, pl.ds(jc * L, L)]
                        plsc.store_scatter(nst_v,
                                           [(jc * L + lane) * SB + w], v16)
                    return 0
                lax.fori_loop(0, SB, tr, 0, unroll=4)

                def upd(jf, _):
                    for g in range(SB // L):
                        sl = pl.ds(g * L, L)
                        col = colb_v[sl]
                        av = avb_v[sl]
                        m = mb_v[sl] != 0
                        idx = jf * sz + col
                        old = plsc.load_gather(slab_v, [idx])
                        nsj = nst_v[pl.ds(jf * SB + g * L, L)]
                        plsc.store_scatter(slab_v, [idx],
                                           av * old + (1.0 - av) * nsj,
                                           mask=m)
                    return 0
                lax.fori_loop(0, MEM_DIM, upd, 0, unroll=2)
                c_sc.wait()
                return 0
            lax.fori_loop(plo // SB, (phi + SB - 1) // SB, sub_batch, 0)

            wrs = [pltpu.async_copy(slab_v.at[pl.ds(j * sz, sz)],
                                    memout_hbm.at[j, pl.ds(off, sz)], s_sl)
                   for j in range(MEM_DIM)]
            for d in wrs:
                d.wait()
            return 0
        return block_pass

    bp1024 = make_block_pass(BLK)
    bp640 = make_block_pass(640)

    def blk_iter(k, _):
        off_std = (wid + 32 * k) * BLK
        off = pl.multiple_of(
            jnp.where(k < 3, off_std, jnp.where(is0, 96 * BLK, wid * BLK)),
            BLK)
        plo = jnp.where(k == 0, p[0],
                        jnp.where(k == 1, p[1],
                                  jnp.where(k == 2, p[2],
                                            jnp.where(is0, p[3], p[0]))))
        phi = jnp.where(k == 0, p[1],
                        jnp.where(k == 1, p[2],
                                  jnp.where(k == 2, p[3],
                                            jnp.where(is0, p[4], p[1]))))
        bp1024(off, plo, phi)
        return 0
    lax.fori_loop(0, 4, blk_iter, 0)

    off3b = pl.multiple_of(jnp.where(is1, 97 * BLK, wid * BLK), BLK)
    bp640(off3b, jnp.where(is1, p[3], p[0]), jnp.where(is1, p[4], p[1]))

    trd = [pltpu.async_copy(tail_hbm.at[r, :],
                            tail_v.at[pl.ds(r * 128, 128)], s_ns)
           for r in range(32)]
    for d in trd:
        d.wait()

    def cnt_grp(g, acc):
        pos = p[3] + g * L + lane
        posc = jnp.clip(pos, p[3], jnp.maximum(p[4] - 1, p[3]))
        li = plsc.load_gather(listc_v, [posc])
        inb = (pos < p[4]) & ((li & 1023) < 640)
        return acc + jnp.sum(inb.astype(jnp.int32))
    ngrp = (p[4] - p[3] + L - 1) // L
    nfront = lax.fori_loop(0, ngrp, cnt_grp, jnp.int32(0))

    def tail_winner(i, _):
        iv = jnp.full((L,), i, jnp.int32)
        li = plsc.load_gather(listc_v, [iv])
        bp = plsc.load_gather(listb_v, [iv])
        c = jnp.clip((li[0] & 1023) - 640, 0, 31)
        bps = bp[0]
        pltpu.sync_copy(ns_hbm.at[pl.ds(bps, 1), :],
                        ns_v.at[pl.ds(0, 1), :])
        nsidx_v[pl.ds(0, L)] = jnp.full((L,), bps, jnp.int32)
        ctb = pltpu.async_copy(ts_hbm.at[nsidx_v.at[pl.ds(0, L)]],
                               tsb_v.at[pl.ds(0, L)], s_ns)
        ctb.wait()
        tsv = tsb_v[pl.ds(0, L)]
        prev = plsc.load_gather(lut_v, [li])
        dt = jnp.maximum(tsv - prev, 0.0)
        av = jnp.exp(-LN2 * dt / HALF_LIFE)
        for jc in range(MEM_DIM // L):
            idx = c * 128 + jc * L + lane
            old = plsc.load_gather(tail_v, [idx])
            ns16 = ns_v[0, pl.ds(jc * L, L)]
            plsc.store_scatter(tail_v, [idx],
                               av * old + (1.0 - av) * ns16)
        return 0
    t_lo = jnp.where(is1, p[3] + nfront, 0)
    t_hi = jnp.where(is1, p[4], 0)
    lax.fori_loop(t_lo, t_hi, tail_winner, 0)

    @pl.when(is1)
    def _():
        pltpu.sync_copy(tail_v, tailout_hbm)


@jax.jit
def kernel(node_ids, new_states, ts, memory, last_update_ts):
    ns128 = jnp.pad(new_states, ((0, 0), (0, 128 - MEM_DIM)))
    tail128 = jnp.pad(lax.slice(memory, (TAIL0, 0), (N_NODES, MEM_DIM)),
                      ((0, 0), (0, 128 - MEM_DIM)))
    mesh = plsc.VectorSubcoreMesh(core_axis_name="c", subcore_axis_name="s",
                                  num_cores=NC, num_subcores=NS)
    f = pl.kernel(
        _sc_body,
        out_type=(jax.ShapeDtypeStruct((MEM_DIM, N_NODES), jnp.float32),
                  jax.ShapeDtypeStruct((N_NODES,), jnp.float32),
                  jax.ShapeDtypeStruct((32 * 128,), jnp.float32)),
        mesh=mesh,
        scratch_types=(
            pltpu.VMEM((BATCH,), jnp.int32),
            pltpu.VMEM((CLAIM,), jnp.float32),
            pltpu.VMEM((CLAIM,), jnp.int32),
            pltpu.VMEM((CLAIM,), jnp.int32),
            pltpu.VMEM((CLAIM,), jnp.int32),
            pltpu.VMEM((MEM_DIM * BLK,), jnp.float32),
            pltpu.VMEM((SB, 128), jnp.float32),
            pltpu.VMEM((MEM_DIM * SB,), jnp.float32),
            pltpu.VMEM((SB,), jnp.int32),
            pltpu.VMEM((SB,), jnp.float32),
            pltpu.VMEM((SB,), jnp.int32),
            pltpu.VMEM((SB,), jnp.int32),
            pltpu.VMEM((SB,), jnp.int32),
            pltpu.VMEM((SB,), jnp.float32),
            pltpu.VMEM((32 * 128,), jnp.float32),
            pltpu.SemaphoreType.DMA,
            pltpu.SemaphoreType.DMA,
            pltpu.SemaphoreType.DMA,
            pltpu.SemaphoreType.DMA,
            pltpu.SemaphoreType.DMA,
        ),
        compiler_params=pltpu.CompilerParams(needs_layout_passes=False,
                                             use_tc_tiling_on_sc=True),
    )
    mo_t, ts_out, tail_out = f(node_ids, ns128, ts, memory.T, last_update_ts,
                               tail128)
    tail_rows = lax.slice(tail_out.reshape(32, 128), (0, 0), (32, MEM_DIM))
    mem_out = lax.dynamic_update_slice(mo_t.T, tail_rows, (TAIL0, 0))
    return mem_out, ts_out
